# baseline (speedup 1.0000x reference)
import functools

import jax
import jax.numpy as jnp
from jax import lax
from jax.experimental import pallas as pl
from jax.experimental.pallas import tpu as pltpu
from jax.experimental.pallas import tpu_sc as plsc

_SEG = 100000
_EPS = 1e-08
_NC = 2
_NS = 16
_NW = _NC * _NS
_SQRT_MAGIC = 0x1FBD1DF5


def _dg(v, idx):
    return lax.gather(
        v,
        idx[:, None],
        lax.GatherDimensionNumbers(
            offset_dims=(), collapsed_slice_dims=(0,), start_index_map=(0,)
        ),
        (1,),
        mode=lax.GatherScatterMode.PROMISE_IN_BOUNDS,
    )


def _lane_iota():
    return lax.iota(jnp.int32, 16)


def _bcast_lane(v, lane):
    return _dg(v, jnp.full((16,), lane, dtype=jnp.int32))


def _inv_std_from_var(var):
    bits = lax.bitcast_convert_type(var, jnp.int32)
    s = lax.bitcast_convert_type((bits >> 1) + _SQRT_MAGIC, jnp.float32)
    s = 0.5 * (s + var / s)
    s = 0.5 * (s + var / s)
    s = 0.5 * (s + var / s)
    return 1.0 / (s + _EPS)


def _build_k1(n_rows, n_seg, tile, ):
    chunk = n_rows // _NW
    assert chunk % tile == 0 and tile % 16 == 0
    groups = tile // 16
    mesh = plsc.VectorSubcoreMesh(
        core_axis_name="c", subcore_axis_name="s",
        num_cores=_NC, num_subcores=_NS,
    )

    def body(x_hbm, ids_hbm, fbm_hbm, fbi_hbm, params_hbm,
             xt, idt, fpb, fib, fib2, pga, pgb, fbmv, fbiv):
        w = lax.axis_index("c") * _NS + lax.axis_index("s")
        base = w * chunk
        stop_row = base + chunk - 1
        lane = _lane_iota()
        shift_idx = jnp.minimum(lane + 1, 15)
        pltpu.sync_copy(fbm_hbm, fbmv)
        pltpu.sync_copy(fbi_hbm, fbiv)
        fbm = fbmv[:]
        fbi = fbiv[:]

        pltpu.sync_copy(
            ids_hbm.at[pl.ds(pl.multiple_of(jnp.maximum(base - 16, 0), 16),
                             16)], pga)
        pltpu.sync_copy(ids_hbm.at[pl.ds(pl.multiple_of(base, 16), 16)], pgb)
        prev_last = _bcast_lane(pga[:], jnp.int32(15))
        cur_first = _bcast_lane(pgb[:], jnp.int32(0))
        neq = jnp.max((prev_last != cur_first).astype(jnp.int32))
        emit0 = jnp.where(w == 0, jnp.int32(1), neq)

        for i in range(8):
            fib2[i, :] = jnp.zeros((16,), jnp.int32)

        zeros = jnp.zeros((16,), jnp.float32)

        def flush_all():
            for i in range(8):
                fib[pl.ds(i * 16, 16)] = fib2[i, :]
            pltpu.sync_copy(fpb, params_hbm.at[fib])
            for i in range(8):
                fib2[i, :] = jnp.zeros((16,), jnp.int32)

        def o_cond(st):
            return (st[6] == 0) & (st[0] < n_rows)

        def o_body(st):
            t0, cnt, sm, sq, emit, cursor, done = st
            t0a = pl.multiple_of(t0, tile)
            pltpu.sync_copy(x_hbm.at[pl.ds(t0a * 16, tile * 16)], xt)
            pltpu.sync_copy(ids_hbm.at[pl.ds(t0a, tile)],
                            idt.at[pl.ds(0, tile)])
            la = pl.multiple_of(
                jnp.minimum(t0a + tile, n_rows - 16), 16)
            pltpu.sync_copy(ids_hbm.at[pl.ds(la, 16)],
                            idt.at[pl.ds(tile, 16)])

            def group_fn(l, gst):
                cnt, sm, sq, emit, cursor, done = gst
                v = idt[pl.ds(pl.multiple_of(l * 16, 16), 16)]
                vn = idt[pl.ds(pl.multiple_of(l * 16 + 16, 16), 16)]
                shifted = _dg(v, shift_idx)
                nxt = jnp.where(lane == 15, _bcast_lane(vn, jnp.int32(0)),
                                shifted)
                grow = t0 + l * 16 + lane
                endm = (v != nxt) | (grow == (n_rows - 1))

                def q_cond(qs):
                    return (qs[0] < 16) & (qs[6] == 0)

                def q_body(qs):
                    q, cnt, sm, sq, emit, cursor, done = qs
                    sel = endm & (lane >= q)
                    ne = jnp.max(plsc.all_reduce_ffs(sel))
                    limit = jnp.minimum(ne + 1, 16)
                    rbase = l * 16

                    def acc(j, c):
                        s_, q_ = c
                        row = xt[pl.ds(pl.multiple_of((rbase + j) * 16, 16),
                                       16)]
                        return (s_ + row, q_ + row * row)

                    sm, sq = lax.fori_loop(q, limit, acc, (sm, sq))
                    cnt = cnt + jnp.full((16,),
                                         (limit - q).astype(jnp.float32))
                    fin = ne < 16
                    safec = jnp.maximum(cnt, 1.0)
                    mean = sm / safec
                    var = jnp.maximum(sq / safec - mean * mean, 0.0)
                    inv = _inv_std_from_var(var)
                    grp = cnt > 1.5
                    pm = jnp.where(grp, mean, fbm)
                    pv = jnp.where(grp, inv, fbi)
                    rid = _bcast_lane(v, jnp.minimum(ne, 15))
                    do_emit = fin & (emit == 1) & (done == 0)

                    @pl.when(do_emit)
                    def _():
                        fpb[cursor, 0:16] = pm
                        fpb[cursor, 16:32] = pv
                        cr = cursor // 16
                        cl = cursor % 16
                        fib2[cr, :] = (fib2[cr, :]
                                       + rid * (lane == cl).astype(jnp.int32))

                    ncur = jnp.where(do_emit, cursor + 1, cursor)

                    @pl.when(ncur == 128)
                    def _():
                        flush_all()

                    cursor = jnp.where(ncur == 128, jnp.int32(0), ncur)
                    rend = t0 + rbase + ne
                    done = jnp.where(fin & (rend >= stop_row),
                                     jnp.int32(1), done)
                    emit = jnp.where(fin, jnp.int32(1), emit)
                    cnt = jnp.where(fin, zeros, cnt)
                    sm = jnp.where(fin, zeros, sm)
                    sq = jnp.where(fin, zeros, sq)
                    return (limit, cnt, sm, sq, emit, cursor, done)

                out = lax.while_loop(
                    q_cond, q_body,
                    (jnp.int32(0), cnt, sm, sq, emit, cursor, done))
                return out[1:]

            cnt, sm, sq, emit, cursor, done = lax.fori_loop(
                0, groups, group_fn, (cnt, sm, sq, emit, cursor, done))
            return (t0 + tile, cnt, sm, sq, emit, cursor, done)

        st = lax.while_loop(
            o_cond, o_body,
            (base, zeros, zeros, zeros, emit0, jnp.int32(0), jnp.int32(0)))
        cursor = st[5]

        @pl.when(cursor > 0)
        def _():
            last = cursor - 1
            lm = fpb[last, 0:16]
            li = fpb[last, 16:32]
            lid = _bcast_lane(fib2[last // 16, :], last % 16)

            def padj(j, carry):
                fpb[j, 0:16] = lm
                fpb[j, 16:32] = li
                fib2[j // 16, :] = (fib2[j // 16, :]
                                    + lid * (lane == (j % 16)).astype(jnp.int32))
                return carry

            lax.fori_loop(cursor, 128, padj, jnp.int32(0))
            flush_all()

    return pl.kernel(
        body,
        out_type=jax.ShapeDtypeStruct((n_seg, 32), jnp.float32),
        mesh=mesh,
        compiler_params=pltpu.CompilerParams(use_tc_tiling_on_sc=False, needs_layout_passes=False),
        scratch_types=[
            pltpu.VMEM((tile * 16,), jnp.float32),
            pltpu.VMEM((tile + 16,), jnp.int32),
            pltpu.VMEM((128, 32), jnp.float32),
            pltpu.VMEM((128,), jnp.int32),
            pltpu.VMEM((8, 16), jnp.int32),
            pltpu.VMEM((16,), jnp.int32),
            pltpu.VMEM((16,), jnp.int32),
            pltpu.VMEM((16,), jnp.float32),
            pltpu.VMEM((16,), jnp.float32),
        ],
    )


def _build_k2(n_rows, n_seg, tile):
    chunk = n_rows // _NW
    batch = 80
    assert chunk % tile == 0 and tile % batch == 0
    nbatch = tile // batch
    ntiles = chunk // tile
    mesh = plsc.VectorSubcoreMesh(
        core_axis_name="c", subcore_axis_name="s",
        num_cores=_NC, num_subcores=_NS,
    )

    def body(x_hbm, ids_hbm, params_hbm, out_hbm, xt, ot, idt, pt, sem):
        w = lax.axis_index("c") * _NS + lax.axis_index("s")
        base = w * chunk

        def tile_fn(k, carry):
            t0 = pl.multiple_of(base + k * tile, tile)
            pltpu.sync_copy(x_hbm.at[pl.ds(t0 * 16, tile * 16)], xt)
            pltpu.sync_copy(ids_hbm.at[pl.ds(t0, tile)], idt)
            copies = [
                pltpu.make_async_copy(
                    params_hbm.at[idt.at[pl.ds(b * batch, batch)]],
                    pt.at[pl.ds(b * batch, batch)],
                    sem,
                )
                for b in range(nbatch)
            ]
            for c in copies:
                c.start()
            for c in copies:
                c.wait()

            def rbody(r, cc):
                ro = pl.multiple_of(r * 16, 16)
                ot[pl.ds(ro, 16)] = ((xt[pl.ds(ro, 16)] - pt[r, 0:16])
                                     * pt[r, 16:32])
                return cc

            lax.fori_loop(0, tile, rbody, jnp.int32(0))
            pltpu.sync_copy(ot, out_hbm.at[pl.ds(t0 * 16, tile * 16)])
            return carry

        lax.fori_loop(0, ntiles, tile_fn, jnp.int32(0))

    return pl.kernel(
        body,
        out_type=jax.ShapeDtypeStruct((n_rows * 16,), jnp.float32),
        mesh=mesh,
        compiler_params=pltpu.CompilerParams(use_tc_tiling_on_sc=False, needs_layout_passes=False),
        scratch_types=[
            pltpu.VMEM((tile * 16,), jnp.float32),
            pltpu.VMEM((tile * 16,), jnp.float32),
            pltpu.VMEM((tile,), jnp.int32),
            pltpu.VMEM((tile, 32), jnp.float32),
            pltpu.SemaphoreType.DMA,
        ],
    )


@functools.partial(jax.jit, static_argnames=())
def _run(x, gid, fbm, fbi):
    n_rows = x.shape[0]
    xf = x.reshape(n_rows * 16)
    k1 = _build_k1(n_rows, _SEG, 2000)
    params = k1(xf, gid, fbm, fbi)
    k2 = _build_k2(n_rows, _SEG, 2000)
    return k2(xf, gid, params).reshape(n_rows, 16)


def kernel(multi_dim_pressures, weights, group_ids, running_mean, running_var):
    x = multi_dim_pressures
    gid = group_ids.astype(jnp.int32)
    fbm = running_mean.astype(jnp.float32)
    fbi = 1.0 / (jnp.sqrt(running_var.astype(jnp.float32)) + _EPS)
    return _run(x, gid, fbm, fbi)


# speedup vs baseline: 1.0399x; 1.0399x over previous
import functools

import jax
import jax.numpy as jnp
from jax import lax
from jax.experimental import pallas as pl
from jax.experimental.pallas import tpu as pltpu
from jax.experimental.pallas import tpu_sc as plsc

_SEG = 100000
_EPS = 1e-08
_NC = 2
_NS = 16
_NW = _NC * _NS
_SQRT_MAGIC = 0x1FBD1DF5


def _dg(v, idx):
    return lax.gather(
        v,
        idx[:, None],
        lax.GatherDimensionNumbers(
            offset_dims=(), collapsed_slice_dims=(0,), start_index_map=(0,)
        ),
        (1,),
        mode=lax.GatherScatterMode.PROMISE_IN_BOUNDS,
    )


def _lane_iota():
    return lax.iota(jnp.int32, 16)


def _bcast_lane(v, lane):
    return _dg(v, jnp.full((16,), lane, dtype=jnp.int32))


def _inv_std_from_var(var):
    bits = lax.bitcast_convert_type(var, jnp.int32)
    s = lax.bitcast_convert_type((bits >> 1) + _SQRT_MAGIC, jnp.float32)
    s = 0.5 * (s + var / s)
    s = 0.5 * (s + var / s)
    s = 0.5 * (s + var / s)
    return 1.0 / (s + _EPS)


def _compiler_params():
    return pltpu.CompilerParams(use_tc_tiling_on_sc=False,
                                needs_layout_passes=False)


_WIN = 2048


def _build_k1(n_rows, n_seg):
    chunk = n_rows // _NW
    assert chunk % 16 == 0 and n_rows % 128 == 0
    nblk = n_rows // 128
    mesh = plsc.VectorSubcoreMesh(
        core_axis_name="c", subcore_axis_name="s",
        num_cores=_NC, num_subcores=_NS,
    )

    def body(x_hbm, ids_hbm, fbm_hbm, fbi_hbm, params_hbm,
             xt, idt, fpb, fib, fib2, pga, pgb, fbmv, fbiv):
        w = lax.axis_index("c") * _NS + lax.axis_index("s")
        base = w * chunk
        stop_row = base + chunk - 1
        lane = _lane_iota()
        shift_idx = jnp.minimum(lane + 1, 15)
        pat = (lane % 8) * 128 + (lane // 8) * (_WIN * 8)
        pltpu.sync_copy(fbm_hbm, fbmv)
        pltpu.sync_copy(fbi_hbm, fbiv)
        fbm = fbmv[:]
        fbi = fbiv[:]

        pltpu.sync_copy(
            ids_hbm.at[pl.ds(pl.multiple_of(jnp.maximum(base - 16, 0), 16),
                             16)], pga)
        pltpu.sync_copy(ids_hbm.at[pl.ds(pl.multiple_of(base, 16), 16)], pgb)
        prev_last = _bcast_lane(pga[:], jnp.int32(15))
        cur_first = _bcast_lane(pgb[:], jnp.int32(0))
        neq = jnp.max((prev_last != cur_first).astype(jnp.int32))
        emit0 = jnp.where(w == 0, jnp.int32(1), neq)

        for i in range(8):
            fib2[i, :] = jnp.zeros((16,), jnp.int32)

        zeros = jnp.zeros((16,), jnp.float32)

        def flush_all():
            for i in range(8):
                fib[pl.ds(i * 16, 16)] = fib2[i, :]
            pltpu.sync_copy(fpb, params_hbm.at[fib])
            for i in range(8):
                fib2[i, :] = jnp.zeros((16,), jnp.int32)

        def stage(win):
            rb = pl.multiple_of((win // 128) * 1024, 1024)
            half = _WIN * 8
            pltpu.sync_copy(x_hbm.at[pl.ds(rb, half)], xt.at[pl.ds(0, half)])
            pltpu.sync_copy(
                x_hbm.at[pl.ds(pl.multiple_of(nblk * 1024 + rb, 1024), half)],
                xt.at[pl.ds(half, half)])
            pltpu.sync_copy(ids_hbm.at[pl.ds(pl.multiple_of(win, 128), _WIN)],
                            idt.at[pl.ds(0, _WIN)])
            la = pl.multiple_of(jnp.minimum(win + _WIN, n_rows - 16), 16)
            pltpu.sync_copy(ids_hbm.at[pl.ds(la, 16)],
                            idt.at[pl.ds(_WIN, 16)])

        win0 = pl.multiple_of(base - base % _WIN, 128)
        stage(win0)

        def o_cond(st):
            return (st[7] == 0) & (st[0] < n_rows)

        def o_body(st):
            grow, win, cnt, sm, sq, emit, cursor, done = st
            need = grow >= win + _WIN
            nwin = jnp.where(need,
                             jnp.minimum(win + _WIN, n_rows - _WIN), win)

            @pl.when(need)
            def _():
                stage(nwin)

            loff = grow - nwin
            v = idt[pl.ds(pl.multiple_of(loff, 16), 16)]
            vn = idt[pl.ds(pl.multiple_of(loff + 16, 16), 16)]
            shifted = _dg(v, shift_idx)
            nxt = jnp.where(lane == 15, _bcast_lane(vn, jnp.int32(0)),
                            shifted)
            endm = (v != nxt) | ((grow + lane) == (n_rows - 1))

            def q_cond(qs):
                return (qs[0] < 16) & (qs[6] == 0)

            def q_body(qs):
                q, cnt, sm, sq, emit, cursor, done = qs
                sel = endm & (lane >= q)
                ne = jnp.max(plsc.all_reduce_ffs(sel))
                limit = jnp.minimum(ne + 1, 16)

                def acc(j, c):
                    s_, q_ = c
                    lr = loff + j
                    o = lr + (lr // 128) * 896
                    row = plsc.load_gather(xt, [pat + jnp.full((16,), o)])
                    return (s_ + row, q_ + row * row)

                sm, sq = lax.fori_loop(q, limit, acc, (sm, sq))
                cnt = cnt + jnp.full((16,), (limit - q).astype(jnp.float32))
                fin = ne < 16
                safec = jnp.maximum(cnt, 1.0)
                mean = sm / safec
                var = jnp.maximum(sq / safec - mean * mean, 0.0)
                inv = _inv_std_from_var(var)
                grp = cnt > 1.5
                pm = jnp.where(grp, mean, fbm)
                pv = jnp.where(grp, inv, fbi)
                rid = _bcast_lane(v, jnp.minimum(ne, 15))
                do_emit = fin & (emit == 1) & (done == 0)

                @pl.when(do_emit)
                def _():
                    fpb[cursor, 0:16] = pm
                    fpb[cursor, 16:32] = pv
                    cr = cursor // 16
                    cl = cursor % 16
                    fib2[cr, :] = (fib2[cr, :]
                                   + rid * (lane == cl).astype(jnp.int32))

                ncur = jnp.where(do_emit, cursor + 1, cursor)

                @pl.when(ncur == 128)
                def _():
                    flush_all()

                cursor = jnp.where(ncur == 128, jnp.int32(0), ncur)
                rend = grow + ne
                done = jnp.where(fin & (rend >= stop_row),
                                 jnp.int32(1), done)
                emit = jnp.where(fin, jnp.int32(1), emit)
                cnt = jnp.where(fin, zeros, cnt)
                sm = jnp.where(fin, zeros, sm)
                sq = jnp.where(fin, zeros, sq)
                return (limit, cnt, sm, sq, emit, cursor, done)

            out = lax.while_loop(
                q_cond, q_body,
                (jnp.int32(0), cnt, sm, sq, emit, cursor, done))
            _, cnt, sm, sq, emit, cursor, done = out
            return (grow + 16, nwin, cnt, sm, sq, emit, cursor, done)

        st = lax.while_loop(
            o_cond, o_body,
            (base, win0, zeros, zeros, zeros, emit0, jnp.int32(0),
             jnp.int32(0)))
        cursor = st[6]

        @pl.when(cursor > 0)
        def _():
            last = cursor - 1
            lm = fpb[last, 0:16]
            li = fpb[last, 16:32]
            lid = _bcast_lane(fib2[last // 16, :], last % 16)

            def padj(j, carry):
                fpb[j, 0:16] = lm
                fpb[j, 16:32] = li
                fib2[j // 16, :] = (fib2[j // 16, :]
                                    + lid * (lane == (j % 16)).astype(jnp.int32))
                return carry

            lax.fori_loop(cursor, 128, padj, jnp.int32(0))
            flush_all()

    return pl.kernel(
        body,
        out_type=jax.ShapeDtypeStruct((n_seg, 32), jnp.float32),
        mesh=mesh,
        compiler_params=_compiler_params(),
        scratch_types=[
            pltpu.VMEM((_WIN * 16,), jnp.float32),
            pltpu.VMEM((_WIN + 16,), jnp.int32),
            pltpu.VMEM((128, 32), jnp.float32),
            pltpu.VMEM((128,), jnp.int32),
            pltpu.VMEM((8, 16), jnp.int32),
            pltpu.VMEM((16,), jnp.int32),
            pltpu.VMEM((16,), jnp.int32),
            pltpu.VMEM((16,), jnp.float32),
            pltpu.VMEM((16,), jnp.float32),
        ],
    )


_T2 = 1024


def _build_k2(n_rows, n_seg):
    assert n_rows % _T2 == 0
    ntiles = n_rows // _T2
    per = (ntiles + _NW - 1) // _NW
    nblk = n_rows // 128
    mesh = plsc.VectorSubcoreMesh(
        core_axis_name="c", subcore_axis_name="s",
        num_cores=_NC, num_subcores=_NS,
    )

    def body(x_hbm, ids_hbm, params_hbm, out_hbm, xt, ot, idt, pt, sem):
        w = lax.axis_index("c") * _NS + lax.axis_index("s")
        lane = _lane_iota()
        half = _T2 * 8

        def tile_fn(i, carry):
            t = w + i * _NW

            @pl.when(t < ntiles)
            def _():
                rb = pl.multiple_of(t * (_T2 * 8), _T2 * 8)
                pltpu.sync_copy(x_hbm.at[pl.ds(rb, half)],
                                xt.at[pl.ds(0, half)])
                pltpu.sync_copy(
                    x_hbm.at[pl.ds(pl.multiple_of(nblk * 1024 + rb, 1024),
                                   half)],
                    xt.at[pl.ds(half, half)])
                pltpu.sync_copy(
                    ids_hbm.at[pl.ds(pl.multiple_of(t * _T2, _T2), _T2)], idt)
                copies = [
                    pltpu.make_async_copy(
                        params_hbm.at[idt.at[pl.ds(b * 128, 128)]],
                        pt.at[pl.ds(b * 128, 128)],
                        sem,
                    )
                    for b in range(_T2 // 128)
                ]
                for c in copies:
                    c.start()
                for c in copies:
                    c.wait()

                def rbody(u, cc):
                    r0 = u * 16
                    rows = r0 + lane
                    lb = u // 8
                    rl = (u % 8) * 16
                    for k in range(16):
                        kh, kl = divmod(k, 8)
                        xo = pl.multiple_of(
                            kh * half + lb * 1024 + kl * 128 + rl, 16)
                        xv = xt[pl.ds(xo, 16)]
                        pm = plsc.load_gather(
                            pt, [rows, jnp.full((16,), k, jnp.int32)])
                        pv = plsc.load_gather(
                            pt, [rows, jnp.full((16,), k + 16, jnp.int32)])
                        ot[pl.ds(xo, 16)] = (xv - pm) * pv
                    return cc

                lax.fori_loop(0, _T2 // 16, rbody, jnp.int32(0))
                pltpu.sync_copy(ot.at[pl.ds(0, half)],
                                out_hbm.at[pl.ds(rb, half)])
                pltpu.sync_copy(
                    ot.at[pl.ds(half, half)],
                    out_hbm.at[pl.ds(pl.multiple_of(nblk * 1024 + rb, 1024),
                                     half)])

            return carry

        lax.fori_loop(0, per, tile_fn, jnp.int32(0))

    return pl.kernel(
        body,
        out_type=jax.ShapeDtypeStruct((n_rows * 16,), jnp.float32),
        mesh=mesh,
        compiler_params=_compiler_params(),
        scratch_types=[
            pltpu.VMEM((_T2 * 16,), jnp.float32),
            pltpu.VMEM((_T2 * 16,), jnp.float32),
            pltpu.VMEM((_T2,), jnp.int32),
            pltpu.VMEM((_T2, 32), jnp.float32),
            pltpu.SemaphoreType.DMA,
        ],
    )


@functools.partial(jax.jit, static_argnames=())
def _run(x, gid, fbm, fbi):
    n_rows = x.shape[0]
    x4 = x.reshape(n_rows // 128, 128, 2, 8).transpose(2, 0, 3, 1)
    xf = x4.reshape(n_rows * 16)
    k1 = _build_k1(n_rows, _SEG)
    params = k1(xf, gid, fbm, fbi)
    k2 = _build_k2(n_rows, _SEG)
    of = k2(xf, gid, params)
    o4 = of.reshape(2, n_rows // 128, 8, 128)
    return o4.transpose(1, 3, 0, 2).reshape(n_rows, 16)


def kernel(multi_dim_pressures, weights, group_ids, running_mean, running_var):
    x = multi_dim_pressures
    gid = group_ids.astype(jnp.int32)
    fbm = running_mean.astype(jnp.float32)
    fbi = 1.0 / (jnp.sqrt(running_var.astype(jnp.float32)) + _EPS)
    return _run(x, gid, fbm, fbi)


# speedup vs baseline: 1.4478x; 1.3923x over previous
import functools

import jax
import jax.numpy as jnp
from jax import lax
from jax.experimental import pallas as pl
from jax.experimental.pallas import tpu as pltpu
from jax.experimental.pallas import tpu_sc as plsc

_SEG = 100000
_EPS = 1e-08
_NC = 2
_NS = 16
_NW = _NC * _NS
_SQRT_MAGIC = 0x1FBD1DF5


def _dg(v, idx):
    return lax.gather(
        v,
        idx[:, None],
        lax.GatherDimensionNumbers(
            offset_dims=(), collapsed_slice_dims=(0,), start_index_map=(0,)
        ),
        (1,),
        mode=lax.GatherScatterMode.PROMISE_IN_BOUNDS,
    )


def _lane_iota():
    return lax.iota(jnp.int32, 16)


def _bcast_lane(v, lane):
    return _dg(v, jnp.full((16,), lane, dtype=jnp.int32))


def _inv_std_from_var(var):
    bits = lax.bitcast_convert_type(var, jnp.int32)
    s = lax.bitcast_convert_type((bits >> 1) + _SQRT_MAGIC, jnp.float32)
    s = 0.5 * (s + var / s)
    s = 0.5 * (s + var / s)
    s = 0.5 * (s + var / s)
    return 1.0 / (s + _EPS)


def _compiler_params():
    return pltpu.CompilerParams(use_tc_tiling_on_sc=False,
                                needs_layout_passes=False)


_WIN = 2048


def _build_k1(n_rows, n_seg):
    chunk = n_rows // _NW
    assert chunk % 16 == 0 and n_rows % 128 == 0
    nblk = n_rows // 128
    mesh = plsc.VectorSubcoreMesh(
        core_axis_name="c", subcore_axis_name="s",
        num_cores=_NC, num_subcores=_NS,
    )

    def body(x_hbm, ids_hbm, fbm_hbm, fbi_hbm, params_hbm,
             xt, idt, fpb, fib, fib2, pga, pgb, fbmv, fbiv):
        w = lax.axis_index("c") * _NS + lax.axis_index("s")
        base = w * chunk
        stop_row = base + chunk - 1
        lane = _lane_iota()
        shift_idx = jnp.minimum(lane + 1, 15)
        pat = (lane % 8) * 128 + (lane // 8) * (_WIN * 8)
        pltpu.sync_copy(fbm_hbm, fbmv)
        pltpu.sync_copy(fbi_hbm, fbiv)
        fbm = fbmv[:]
        fbi = fbiv[:]

        pltpu.sync_copy(
            ids_hbm.at[pl.ds(pl.multiple_of(jnp.maximum(base - 16, 0), 16),
                             16)], pga)
        pltpu.sync_copy(ids_hbm.at[pl.ds(pl.multiple_of(base, 16), 16)], pgb)
        prev_last = _bcast_lane(pga[:], jnp.int32(15))
        cur_first = _bcast_lane(pgb[:], jnp.int32(0))
        neq = jnp.max((prev_last != cur_first).astype(jnp.int32))
        emit0 = jnp.where(w == 0, jnp.int32(1), neq)

        for i in range(8):
            fib2[i, :] = jnp.zeros((16,), jnp.int32)

        zeros = jnp.zeros((16,), jnp.float32)

        def flush_all():
            for i in range(8):
                fib[pl.ds(i * 16, 16)] = fib2[i, :]
            pltpu.sync_copy(fpb, params_hbm.at[fib])
            for i in range(8):
                fib2[i, :] = jnp.zeros((16,), jnp.int32)

        def stage(win):
            rb = pl.multiple_of((win // 128) * 1024, 1024)
            half = _WIN * 8
            pltpu.sync_copy(x_hbm.at[pl.ds(rb, half)], xt.at[pl.ds(0, half)])
            pltpu.sync_copy(
                x_hbm.at[pl.ds(pl.multiple_of(nblk * 1024 + rb, 1024), half)],
                xt.at[pl.ds(half, half)])
            pltpu.sync_copy(ids_hbm.at[pl.ds(pl.multiple_of(win, 128), _WIN)],
                            idt.at[pl.ds(0, _WIN)])
            la = pl.multiple_of(jnp.minimum(win + _WIN, n_rows - 16), 16)
            pltpu.sync_copy(ids_hbm.at[pl.ds(la, 16)],
                            idt.at[pl.ds(_WIN, 16)])

        win0 = pl.multiple_of(base - base % _WIN, 128)
        stage(win0)

        def o_cond(st):
            return (st[7] == 0) & (st[0] < n_rows)

        def o_body(st):
            grow, win, cnt, sm, sq, emit, cursor, done = st
            need = grow >= win + _WIN
            nwin = jnp.where(need,
                             jnp.minimum(win + _WIN, n_rows - _WIN), win)

            @pl.when(need)
            def _():
                stage(nwin)

            loff = grow - nwin
            v = idt[pl.ds(pl.multiple_of(loff, 16), 16)]
            vn = idt[pl.ds(pl.multiple_of(loff + 16, 16), 16)]
            shifted = _dg(v, shift_idx)
            nxt = jnp.where(lane == 15, _bcast_lane(vn, jnp.int32(0)),
                            shifted)
            endm = (v != nxt) | ((grow + lane) == (n_rows - 1))

            def q_cond(qs):
                return (qs[0] < 16) & (qs[6] == 0)

            def q_body(qs):
                q, cnt, sm, sq, emit, cursor, done = qs
                sel = endm & (lane >= q)
                ne = jnp.max(plsc.all_reduce_ffs(sel))
                limit = jnp.minimum(ne + 1, 16)

                def acc(j, c):
                    s_, q_ = c
                    lr = loff + j
                    o = lr + (lr // 128) * 896
                    row = plsc.load_gather(xt, [pat + jnp.full((16,), o)])
                    return (s_ + row, q_ + row * row)

                sm, sq = lax.fori_loop(q, limit, acc, (sm, sq))
                cnt = cnt + jnp.full((16,), (limit - q).astype(jnp.float32))
                fin = ne < 16
                safec = jnp.maximum(cnt, 1.0)
                mean = sm / safec
                var = jnp.maximum(sq / safec - mean * mean, 0.0)
                inv = _inv_std_from_var(var)
                grp = cnt > 1.5
                pm = jnp.where(grp, mean, fbm)
                pv = jnp.where(grp, inv, fbi)
                rid = _bcast_lane(v, jnp.minimum(ne, 15))
                do_emit = fin & (emit == 1) & (done == 0)

                @pl.when(do_emit)
                def _():
                    fpb[cursor, 0:16] = pm
                    fpb[cursor, 16:32] = pv
                    cr = cursor // 16
                    cl = cursor % 16
                    fib2[cr, :] = (fib2[cr, :]
                                   + rid * (lane == cl).astype(jnp.int32))

                ncur = jnp.where(do_emit, cursor + 1, cursor)

                @pl.when(ncur == 128)
                def _():
                    flush_all()

                cursor = jnp.where(ncur == 128, jnp.int32(0), ncur)
                rend = grow + ne
                done = jnp.where(fin & (rend >= stop_row),
                                 jnp.int32(1), done)
                emit = jnp.where(fin, jnp.int32(1), emit)
                cnt = jnp.where(fin, zeros, cnt)
                sm = jnp.where(fin, zeros, sm)
                sq = jnp.where(fin, zeros, sq)
                return (limit, cnt, sm, sq, emit, cursor, done)

            out = lax.while_loop(
                q_cond, q_body,
                (jnp.int32(0), cnt, sm, sq, emit, cursor, done))
            _, cnt, sm, sq, emit, cursor, done = out
            return (grow + 16, nwin, cnt, sm, sq, emit, cursor, done)

        st = lax.while_loop(
            o_cond, o_body,
            (base, win0, zeros, zeros, zeros, emit0, jnp.int32(0),
             jnp.int32(0)))
        cursor = st[6]

        @pl.when(cursor > 0)
        def _():
            last = cursor - 1
            lm = fpb[last, 0:16]
            li = fpb[last, 16:32]
            lid = _bcast_lane(fib2[last // 16, :], last % 16)

            def padj(j, carry):
                fpb[j, 0:16] = lm
                fpb[j, 16:32] = li
                fib2[j // 16, :] = (fib2[j // 16, :]
                                    + lid * (lane == (j % 16)).astype(jnp.int32))
                return carry

            lax.fori_loop(cursor, 128, padj, jnp.int32(0))
            flush_all()

    return pl.kernel(
        body,
        out_type=jax.ShapeDtypeStruct((n_seg, 32), jnp.float32),
        mesh=mesh,
        compiler_params=_compiler_params(),
        scratch_types=[
            pltpu.VMEM((_WIN * 16,), jnp.float32),
            pltpu.VMEM((_WIN + 16,), jnp.int32),
            pltpu.VMEM((128, 32), jnp.float32),
            pltpu.VMEM((128,), jnp.int32),
            pltpu.VMEM((8, 16), jnp.int32),
            pltpu.VMEM((16,), jnp.int32),
            pltpu.VMEM((16,), jnp.int32),
            pltpu.VMEM((16,), jnp.float32),
            pltpu.VMEM((16,), jnp.float32),
        ],
    )


_T2 = 1024


def _build_k2(n_rows, n_seg):
    assert n_rows % _T2 == 0
    ntiles = n_rows // _T2
    per = (ntiles + _NW - 1) // _NW
    nblk = n_rows // 128
    mesh = plsc.VectorSubcoreMesh(
        core_axis_name="c", subcore_axis_name="s",
        num_cores=_NC, num_subcores=_NS,
    )

    def body(x_hbm, ids_hbm, params_hbm, out_hbm, xt, ot, idt, pt, sem):
        w = lax.axis_index("c") * _NS + lax.axis_index("s")
        lane = _lane_iota()
        half = _T2 * 8

        def tile_fn(i, carry):
            t = w + i * _NW

            @pl.when(t < ntiles)
            def _():
                rb = pl.multiple_of(t * (_T2 * 8), _T2 * 8)
                pltpu.sync_copy(x_hbm.at[pl.ds(rb, half)],
                                xt.at[pl.ds(0, half)])
                pltpu.sync_copy(
                    x_hbm.at[pl.ds(pl.multiple_of(nblk * 1024 + rb, 1024),
                                   half)],
                    xt.at[pl.ds(half, half)])
                pltpu.sync_copy(
                    ids_hbm.at[pl.ds(pl.multiple_of(t * _T2, _T2), _T2)], idt)
                copies = [
                    pltpu.make_async_copy(
                        params_hbm.at[idt.at[pl.ds(b * 128, 128)]],
                        pt.at[pl.ds(b * 128, 128)],
                        sem,
                    )
                    for b in range(_T2 // 128)
                ]
                for c in copies:
                    c.start()
                for c in copies:
                    c.wait()

                shift_idx = jnp.minimum(lane + 1, 15)
                zeros = jnp.zeros((16,), jnp.float32)

                def rbody(u, cc):
                    r0 = u * 16
                    lb = u // 8
                    rl = (u % 8) * 16
                    v = idt[pl.ds(pl.multiple_of(r0, 16), 16)]
                    endm = v != _dg(v, shift_idx)
                    xs = []
                    for k in range(16):
                        kh, kl = divmod(k, 8)
                        xo = pl.multiple_of(
                            kh * half + lb * 1024 + kl * 128 + rl, 16)
                        xs.append(xt[pl.ds(xo, 16)])

                    def q_cond(qs):
                        return qs[0] < 16

                    def q_body(qs):
                        q = qs[0]
                        acc = qs[1:]
                        sel = endm & (lane >= q)
                        ne = jnp.max(plsc.all_reduce_ffs(sel))
                        limit = jnp.minimum(ne + 1, 16)
                        pm = pt[r0 + q, 0:16]
                        pv = pt[r0 + q, 16:32]
                        mask = (lane >= q) & (lane < limit)
                        nacc = []
                        for k in range(16):
                            val = ((xs[k] - _bcast_lane(pm, k))
                                   * _bcast_lane(pv, k))
                            nacc.append(jnp.where(mask, val, acc[k]))
                        return (limit,) + tuple(nacc)

                    res = lax.while_loop(q_cond, q_body,
                                         (jnp.int32(0),) + (zeros,) * 16)
                    for k in range(16):
                        kh, kl = divmod(k, 8)
                        xo = pl.multiple_of(
                            kh * half + lb * 1024 + kl * 128 + rl, 16)
                        ot[pl.ds(xo, 16)] = res[1 + k]
                    return cc

                lax.fori_loop(0, _T2 // 16, rbody, jnp.int32(0))
                pltpu.sync_copy(ot.at[pl.ds(0, half)],
                                out_hbm.at[pl.ds(rb, half)])
                pltpu.sync_copy(
                    ot.at[pl.ds(half, half)],
                    out_hbm.at[pl.ds(pl.multiple_of(nblk * 1024 + rb, 1024),
                                     half)])

            return carry

        lax.fori_loop(0, per, tile_fn, jnp.int32(0))

    return pl.kernel(
        body,
        out_type=jax.ShapeDtypeStruct((n_rows * 16,), jnp.float32),
        mesh=mesh,
        compiler_params=_compiler_params(),
        scratch_types=[
            pltpu.VMEM((_T2 * 16,), jnp.float32),
            pltpu.VMEM((_T2 * 16,), jnp.float32),
            pltpu.VMEM((_T2,), jnp.int32),
            pltpu.VMEM((_T2, 32), jnp.float32),
            pltpu.SemaphoreType.DMA,
        ],
    )


@functools.partial(jax.jit, static_argnames=())
def _run(x, gid, fbm, fbi):
    n_rows = x.shape[0]
    x4 = x.reshape(n_rows // 128, 128, 2, 8).transpose(2, 0, 3, 1)
    xf = x4.reshape(n_rows * 16)
    k1 = _build_k1(n_rows, _SEG)
    params = k1(xf, gid, fbm, fbi)
    k2 = _build_k2(n_rows, _SEG)
    of = k2(xf, gid, params)
    o4 = of.reshape(2, n_rows // 128, 8, 128)
    return o4.transpose(1, 3, 0, 2).reshape(n_rows, 16)


def kernel(multi_dim_pressures, weights, group_ids, running_mean, running_var):
    x = multi_dim_pressures
    gid = group_ids.astype(jnp.int32)
    fbm = running_mean.astype(jnp.float32)
    fbi = 1.0 / (jnp.sqrt(running_var.astype(jnp.float32)) + _EPS)
    return _run(x, gid, fbm, fbi)


# speedup vs baseline: 1.9049x; 1.3157x over previous
import functools

import jax
import jax.numpy as jnp
from jax import lax
from jax.experimental import pallas as pl
from jax.experimental.pallas import tpu as pltpu
from jax.experimental.pallas import tpu_sc as plsc

_SEG = 100000
_EPS = 1e-08
_NC = 2
_NS = 16
_NW = _NC * _NS
_SQRT_MAGIC = 0x1FBD1DF5


def _dg(v, idx):
    return lax.gather(
        v,
        idx[:, None],
        lax.GatherDimensionNumbers(
            offset_dims=(), collapsed_slice_dims=(0,), start_index_map=(0,)
        ),
        (1,),
        mode=lax.GatherScatterMode.PROMISE_IN_BOUNDS,
    )


def _lane_iota():
    return lax.iota(jnp.int32, 16)


def _bcast_lane(v, lane):
    return _dg(v, jnp.full((16,), lane, dtype=jnp.int32))


def _inv_std_from_var(var):
    bits = lax.bitcast_convert_type(var, jnp.int32)
    s = lax.bitcast_convert_type((bits >> 1) + _SQRT_MAGIC, jnp.float32)
    s = 0.5 * (s + var / s)
    s = 0.5 * (s + var / s)
    s = 0.5 * (s + var / s)
    return 1.0 / (s + _EPS)


def _transpose16(vs, lane):
    cur = list(vs)
    for d in (1, 2, 4, 8):
        idm = (lane - d) & 15
        idp = (lane + d) & 15
        sel = (lane & d) == 0
        nxt = list(cur)
        for i in range(16):
            if i & d == 0:
                a, b = cur[i], cur[i + d]
                nxt[i] = jnp.where(sel, a, _dg(b, idm))
                nxt[i + d] = jnp.where(sel, _dg(a, idp), b)
        cur = nxt
    return cur


def _compiler_params():
    return pltpu.CompilerParams(use_tc_tiling_on_sc=False,
                                needs_layout_passes=False)


_WIN = 2048


def _build_k1(n_rows, n_seg):
    chunk = n_rows // _NW
    assert chunk % 16 == 0 and n_rows % 128 == 0
    nblk = n_rows // 128
    mesh = plsc.VectorSubcoreMesh(
        core_axis_name="c", subcore_axis_name="s",
        num_cores=_NC, num_subcores=_NS,
    )

    def body(x_hbm, ids_hbm, fbm_hbm, fbi_hbm, params_hbm,
             xt, idt, fpb, fib, fib2, pga, pgb, fbmv, fbiv):
        w = lax.axis_index("c") * _NS + lax.axis_index("s")
        base = w * chunk
        stop_row = base + chunk - 1
        lane = _lane_iota()
        shift_idx = jnp.minimum(lane + 1, 15)
        pat = (lane % 8) * 128 + (lane // 8) * (_WIN * 8)
        pltpu.sync_copy(fbm_hbm, fbmv)
        pltpu.sync_copy(fbi_hbm, fbiv)
        fbm = fbmv[:]
        fbi = fbiv[:]

        pltpu.sync_copy(
            ids_hbm.at[pl.ds(pl.multiple_of(jnp.maximum(base - 16, 0), 16),
                             16)], pga)
        pltpu.sync_copy(ids_hbm.at[pl.ds(pl.multiple_of(base, 16), 16)], pgb)
        prev_last = _bcast_lane(pga[:], jnp.int32(15))
        cur_first = _bcast_lane(pgb[:], jnp.int32(0))
        neq = jnp.max((prev_last != cur_first).astype(jnp.int32))
        emit0 = jnp.where(w == 0, jnp.int32(1), neq)

        for i in range(8):
            fib2[i, :] = jnp.zeros((16,), jnp.int32)

        zeros = jnp.zeros((16,), jnp.float32)

        def flush_all():
            for i in range(8):
                fib[pl.ds(i * 16, 16)] = fib2[i, :]
            pltpu.sync_copy(fpb, params_hbm.at[fib])
            for i in range(8):
                fib2[i, :] = jnp.zeros((16,), jnp.int32)

        def stage(win):
            rb = pl.multiple_of((win // 128) * 1024, 1024)
            half = _WIN * 8
            pltpu.sync_copy(x_hbm.at[pl.ds(rb, half)], xt.at[pl.ds(0, half)])
            pltpu.sync_copy(
                x_hbm.at[pl.ds(pl.multiple_of(nblk * 1024 + rb, 1024), half)],
                xt.at[pl.ds(half, half)])
            pltpu.sync_copy(ids_hbm.at[pl.ds(pl.multiple_of(win, 128), _WIN)],
                            idt.at[pl.ds(0, _WIN)])
            la = pl.multiple_of(jnp.minimum(win + _WIN, n_rows - 16), 16)
            pltpu.sync_copy(ids_hbm.at[pl.ds(la, 16)],
                            idt.at[pl.ds(_WIN, 16)])

        win0 = pl.multiple_of(base - base % _WIN, 128)
        stage(win0)

        def o_cond(st):
            return (st[7] == 0) & (st[0] < n_rows)

        def o_body(st):
            grow, win, cnt, sm, sq, emit, cursor, done = st
            need = grow >= win + _WIN
            nwin = jnp.where(need,
                             jnp.minimum(win + _WIN, n_rows - _WIN), win)

            @pl.when(need)
            def _():
                stage(nwin)

            loff = grow - nwin
            v = idt[pl.ds(pl.multiple_of(loff, 16), 16)]
            vn = idt[pl.ds(pl.multiple_of(loff + 16, 16), 16)]
            shifted = _dg(v, shift_idx)
            nxt = jnp.where(lane == 15, _bcast_lane(vn, jnp.int32(0)),
                            shifted)
            endm = (v != nxt) | ((grow + lane) == (n_rows - 1))

            o2 = loff + (loff // 128) * 896
            xs = []
            for k in range(16):
                kh, kl = divmod(k, 8)
                xs.append(xt[pl.ds(pl.multiple_of(
                    kh * (_WIN * 8) + kl * 128 + o2, 16), 16)])
            ts = _transpose16(xs, lane)
            t2 = [t * t for t in ts]

            def q_cond(qs):
                return (qs[0] < 16) & (qs[6] == 0)

            def q_body(qs):
                q, cnt, sm, sq, emit, cursor, done = qs
                sel = endm & (lane >= q)
                ne = jnp.max(plsc.all_reduce_ffs(sel))
                limit = jnp.minimum(ne + 1, 16)

                zf = jnp.zeros((16,), jnp.float32)
                for r in range(16):
                    mr = (r >= q) & (r < limit)
                    sm = sm + jnp.where(mr, ts[r], zf)
                    sq = sq + jnp.where(mr, t2[r], zf)
                cnt = cnt + jnp.full((16,), (limit - q).astype(jnp.float32))
                fin = ne < 16
                safec = jnp.maximum(cnt, 1.0)
                mean = sm / safec
                var = jnp.maximum(sq / safec - mean * mean, 0.0)
                inv = _inv_std_from_var(var)
                grp = cnt > 1.5
                pm = jnp.where(grp, mean, fbm)
                pv = jnp.where(grp, inv, fbi)
                rid = _bcast_lane(v, jnp.minimum(ne, 15))
                do_emit = fin & (emit == 1) & (done == 0)

                @pl.when(do_emit)
                def _():
                    fpb[cursor, 0:16] = pm
                    fpb[cursor, 16:32] = pv
                    cr = cursor // 16
                    cl = cursor % 16
                    fib2[cr, :] = (fib2[cr, :]
                                   + rid * (lane == cl).astype(jnp.int32))

                ncur = jnp.where(do_emit, cursor + 1, cursor)

                @pl.when(ncur == 128)
                def _():
                    flush_all()

                cursor = jnp.where(ncur == 128, jnp.int32(0), ncur)
                rend = grow + ne
                done = jnp.where(fin & (rend >= stop_row),
                                 jnp.int32(1), done)
                emit = jnp.where(fin, jnp.int32(1), emit)
                cnt = jnp.where(fin, zeros, cnt)
                sm = jnp.where(fin, zeros, sm)
                sq = jnp.where(fin, zeros, sq)
                return (limit, cnt, sm, sq, emit, cursor, done)

            out = lax.while_loop(
                q_cond, q_body,
                (jnp.int32(0), cnt, sm, sq, emit, cursor, done))
            _, cnt, sm, sq, emit, cursor, done = out
            return (grow + 16, nwin, cnt, sm, sq, emit, cursor, done)

        st = lax.while_loop(
            o_cond, o_body,
            (base, win0, zeros, zeros, zeros, emit0, jnp.int32(0),
             jnp.int32(0)))
        cursor = st[6]

        @pl.when(cursor > 0)
        def _():
            last = cursor - 1
            lm = fpb[last, 0:16]
            li = fpb[last, 16:32]
            lid = _bcast_lane(fib2[last // 16, :], last % 16)

            def padj(j, carry):
                fpb[j, 0:16] = lm
                fpb[j, 16:32] = li
                fib2[j // 16, :] = (fib2[j // 16, :]
                                    + lid * (lane == (j % 16)).astype(jnp.int32))
                return carry

            lax.fori_loop(cursor, 128, padj, jnp.int32(0))
            flush_all()

    return pl.kernel(
        body,
        out_type=jax.ShapeDtypeStruct((n_seg, 32), jnp.float32),
        mesh=mesh,
        compiler_params=_compiler_params(),
        scratch_types=[
            pltpu.VMEM((_WIN * 16,), jnp.float32),
            pltpu.VMEM((_WIN + 16,), jnp.int32),
            pltpu.VMEM((128, 32), jnp.float32),
            pltpu.VMEM((128,), jnp.int32),
            pltpu.VMEM((8, 16), jnp.int32),
            pltpu.VMEM((16,), jnp.int32),
            pltpu.VMEM((16,), jnp.int32),
            pltpu.VMEM((16,), jnp.float32),
            pltpu.VMEM((16,), jnp.float32),
        ],
    )


_T2 = 1024


def _build_k2(n_rows, n_seg):
    assert n_rows % _T2 == 0
    ntiles = n_rows // _T2
    per = (ntiles + _NW - 1) // _NW
    nblk = n_rows // 128
    mesh = plsc.VectorSubcoreMesh(
        core_axis_name="c", subcore_axis_name="s",
        num_cores=_NC, num_subcores=_NS,
    )

    def body(x_hbm, ids_hbm, params_hbm, out_hbm, xt, ot, idt, pt, sem):
        w = lax.axis_index("c") * _NS + lax.axis_index("s")
        lane = _lane_iota()
        half = _T2 * 8

        def tile_fn(i, carry):
            t = w + i * _NW

            @pl.when(t < ntiles)
            def _():
                rb = pl.multiple_of(t * (_T2 * 8), _T2 * 8)
                pltpu.sync_copy(x_hbm.at[pl.ds(rb, half)],
                                xt.at[pl.ds(0, half)])
                pltpu.sync_copy(
                    x_hbm.at[pl.ds(pl.multiple_of(nblk * 1024 + rb, 1024),
                                   half)],
                    xt.at[pl.ds(half, half)])
                pltpu.sync_copy(
                    ids_hbm.at[pl.ds(pl.multiple_of(t * _T2, _T2), _T2)], idt)
                copies = [
                    pltpu.make_async_copy(
                        params_hbm.at[idt.at[pl.ds(b * 128, 128)]],
                        pt.at[pl.ds(b * 128, 128)],
                        sem,
                    )
                    for b in range(_T2 // 128)
                ]
                for c in copies:
                    c.start()
                for c in copies:
                    c.wait()

                shift_idx = jnp.minimum(lane + 1, 15)
                zeros = jnp.zeros((16,), jnp.float32)

                def rbody(u, cc):
                    r0 = u * 16
                    lb = u // 8
                    rl = (u % 8) * 16
                    v = idt[pl.ds(pl.multiple_of(r0, 16), 16)]
                    endm = v != _dg(v, shift_idx)
                    xs = []
                    for k in range(16):
                        kh, kl = divmod(k, 8)
                        xo = pl.multiple_of(
                            kh * half + lb * 1024 + kl * 128 + rl, 16)
                        xs.append(xt[pl.ds(xo, 16)])

                    def q_cond(qs):
                        return qs[0] < 16

                    def q_body(qs):
                        q = qs[0]
                        acc = qs[1:]
                        sel = endm & (lane >= q)
                        ne = jnp.max(plsc.all_reduce_ffs(sel))
                        limit = jnp.minimum(ne + 1, 16)
                        pm = pt[r0 + q, 0:16]
                        pv = pt[r0 + q, 16:32]
                        mask = (lane >= q) & (lane < limit)
                        nacc = []
                        for k in range(16):
                            val = ((xs[k] - _bcast_lane(pm, k))
                                   * _bcast_lane(pv, k))
                            nacc.append(jnp.where(mask, val, acc[k]))
                        return (limit,) + tuple(nacc)

                    res = lax.while_loop(q_cond, q_body,
                                         (jnp.int32(0),) + (zeros,) * 16)
                    for k in range(16):
                        kh, kl = divmod(k, 8)
                        xo = pl.multiple_of(
                            kh * half + lb * 1024 + kl * 128 + rl, 16)
                        ot[pl.ds(xo, 16)] = res[1 + k]
                    return cc

                lax.fori_loop(0, _T2 // 16, rbody, jnp.int32(0))
                pltpu.sync_copy(ot.at[pl.ds(0, half)],
                                out_hbm.at[pl.ds(rb, half)])
                pltpu.sync_copy(
                    ot.at[pl.ds(half, half)],
                    out_hbm.at[pl.ds(pl.multiple_of(nblk * 1024 + rb, 1024),
                                     half)])

            return carry

        lax.fori_loop(0, per, tile_fn, jnp.int32(0))

    return pl.kernel(
        body,
        out_type=jax.ShapeDtypeStruct((n_rows * 16,), jnp.float32),
        mesh=mesh,
        compiler_params=_compiler_params(),
        scratch_types=[
            pltpu.VMEM((_T2 * 16,), jnp.float32),
            pltpu.VMEM((_T2 * 16,), jnp.float32),
            pltpu.VMEM((_T2,), jnp.int32),
            pltpu.VMEM((_T2, 32), jnp.float32),
            pltpu.SemaphoreType.DMA,
        ],
    )


@functools.partial(jax.jit, static_argnames=())
def _run(x, gid, fbm, fbi):
    n_rows = x.shape[0]
    x4 = x.reshape(n_rows // 128, 128, 2, 8).transpose(2, 0, 3, 1)
    xf = x4.reshape(n_rows * 16)
    k1 = _build_k1(n_rows, _SEG)
    params = k1(xf, gid, fbm, fbi)
    k2 = _build_k2(n_rows, _SEG)
    of = k2(xf, gid, params)
    o4 = of.reshape(2, n_rows // 128, 8, 128)
    return o4.transpose(1, 3, 0, 2).reshape(n_rows, 16)


def kernel(multi_dim_pressures, weights, group_ids, running_mean, running_var):
    x = multi_dim_pressures
    gid = group_ids.astype(jnp.int32)
    fbm = running_mean.astype(jnp.float32)
    fbi = 1.0 / (jnp.sqrt(running_var.astype(jnp.float32)) + _EPS)
    return _run(x, gid, fbm, fbi)
